# baseline (speedup 1.0000x reference)
import functools

import jax
import jax.numpy as jnp
from jax import lax
from jax.experimental import pallas as pl
from jax.experimental.pallas import tpu as pltpu
from jax.experimental.pallas import tpu_sc as plsc

N_NODES = 10000
DIM = 128
E_EDGES = 320000

NC = 2
NS = 16
NW = NC * NS

N_PAD = 10240
RPT = N_PAD // NS
K = 128
EPT = 10240
CHUNKS = EPT // K
KA = 64
ACH = EPT // KA
NPH = 4
AHALF = ACH // NPH
RING = 4
E_PAD = NW * EPT

_mesh = plsc.VectorSubcoreMesh(core_axis_name="c", subcore_axis_name="s")


def _deg_body(src_hbm, zeros_hbm, ones_hbm, deg_out, src_v, ones_v, acc):
    c = lax.axis_index("c")
    s = lax.axis_index("s")
    w = c * NS + s
    pltpu.sync_copy(zeros_hbm, acc.at[pl.ds(s * RPT, RPT)])
    pltpu.sync_copy(ones_hbm, ones_v)
    plsc.subcore_barrier()

    def body(j, carry):
        pltpu.sync_copy(ones_v, acc.at[src_v.at[j]], add=True)
        return carry

    for h in (0, 1):
        pltpu.sync_copy(src_hbm.at[w, pl.ds(h * (CHUNKS // 2), CHUNKS // 2)],
                        src_v)
        lax.fori_loop(0, CHUNKS // 2, body, 0)
    plsc.subcore_barrier()
    pltpu.sync_copy(acc.at[pl.ds(s * RPT, RPT)],
                    deg_out.at[c, pl.ds(s * RPT, RPT)])


_deg_kernel = functools.partial(
    pl.kernel,
    out_type=jax.ShapeDtypeStruct((NC, N_PAD), jnp.float32),
    mesh=_mesh,
    scratch_types=[
        pltpu.VMEM((CHUNKS // 2, K), jnp.int32),
        pltpu.VMEM((K,), jnp.float32),
        pltpu.VMEM_SHARED((N_PAD,), jnp.float32),
    ],
)(_deg_body)


def _agg_body(src_hbm, dst_hbm, xn_hbm, zeros_hbm, part_out,
              src_v, dst_v, r0, r1, r2, r3,
              acc, g0, g1, g2, g3, s0, s1, s2, s3):
    rows = [r0, r1, r2, r3]
    gsem = [g0, g1, g2, g3]
    ssem = [s0, s1, s2, s3]
    c = lax.axis_index("c")
    s = lax.axis_index("s")
    w = c * NS + s
    pltpu.sync_copy(zeros_hbm, acc.at[pl.ds(s * RPT, RPT)])
    plsc.subcore_barrier()

    def gat(j, buf, sem):
        return pltpu.make_async_copy(xn_hbm.at[src_v.at[j]], buf, sem)

    def sca(j, buf, sem):
        return pltpu.make_async_copy(buf, acc.at[dst_v.at[j]], sem)

    def process_half(h):
        pltpu.sync_copy(src_hbm.at[w, pl.ds(h * AHALF, AHALF)], src_v)
        pltpu.sync_copy(dst_hbm.at[w, pl.ds(h * AHALF, AHALF)], dst_v)
        gat(0, rows[0], gsem[0]).start()
        gat(1, rows[1], gsem[1]).start()
        for u in (0, 1):
            gat(u, rows[u], gsem[u]).wait()
            sca(u, rows[u], ssem[u]).start(add=True)
            gat(u + 2, rows[u + 2], gsem[u + 2]).start()

        def body(t, carry):
            j0 = 4 * t + 2
            for u in range(RING):
                j = j0 + u
                b = (2 + u) % RING
                pb = (u) % RING
                gat(j, rows[b], gsem[b]).wait()
                sca(j, rows[b], ssem[b]).start(add=True)
                sca(j - 2, rows[pb], ssem[pb]).wait()
                gat(j + 2, rows[pb], gsem[pb]).start()
            return carry

        lax.fori_loop(0, (AHALF - 4) // 4, body, 0)
        for u in (0, 1):
            j = AHALF - 2 + u
            b = j % RING
            gat(j, rows[b], gsem[b]).wait()
            sca(j, rows[b], ssem[b]).start(add=True)
            sca(j - 2, rows[(j - 2) % RING], ssem[(j - 2) % RING]).wait()
        for u in (0, 1):
            j = AHALF - 2 + u
            sca(j, rows[j % RING], ssem[j % RING]).wait()

    for h in range(NPH):
        process_half(h)
    plsc.subcore_barrier()
    pltpu.sync_copy(acc.at[pl.ds(s * RPT, RPT)],
                    part_out.at[c, pl.ds(s * RPT, RPT)])


_agg_kernel = functools.partial(
    pl.kernel,
    out_type=jax.ShapeDtypeStruct((NC, N_PAD, DIM), jnp.float32),
    mesh=_mesh,
    scratch_types=(
        [pltpu.VMEM((AHALF, KA), jnp.int32),
         pltpu.VMEM((AHALF, KA), jnp.int32)]
        + [pltpu.VMEM((KA, DIM), jnp.float32)] * RING
        + [pltpu.VMEM_SHARED((N_PAD, DIM), jnp.float32)]
        + [pltpu.SemaphoreType.DMA] * (2 * RING)
    ),
)(_agg_body)


BN = 1000


def _scale_body(deg_ref, x_ref, xn_ref):
    d = deg_ref[0] + deg_ref[1]
    norm = lax.rsqrt(jnp.maximum(d, 1.0))
    xn_ref[...] = x_ref[...] * norm


def _out_body(part_ref, deg_ref, w1_ref, b1_ref, w2_ref, b2_ref, out_ref):
    p = part_ref[0] + part_ref[1]
    d = deg_ref[0] + deg_ref[1]
    norm = lax.rsqrt(jnp.maximum(d, 1.0))
    h = p * norm
    h = jnp.dot(h, w1_ref[...], preferred_element_type=jnp.float32)
    h = jnp.maximum(h + b1_ref[...], 0.0)
    out_ref[...] = (jnp.dot(h, w2_ref[...], preferred_element_type=jnp.float32)
                    + b2_ref[...])


def kernel(x, edge_index, W1, b1, W2, b2):
    src = edge_index[0]
    dst = edge_index[1]
    pad = E_PAD - E_EDGES
    pad_hi = (N_NODES + jnp.arange(pad, dtype=jnp.int32)
              % (N_PAD - N_NODES)).astype(jnp.int32)
    pad_lo = (jnp.arange(pad, dtype=jnp.int32) % N_NODES).astype(jnp.int32)
    src_deg = jnp.concatenate([src, pad_hi]).reshape(NW, CHUNKS, K)
    src_gat = jnp.concatenate([src, pad_lo]).reshape(NW, ACH, KA)
    dst_sc = jnp.concatenate([dst, pad_hi]).reshape(NW, ACH, KA)

    zeros_1d = jnp.zeros((RPT,), jnp.float32)
    ones_k = jnp.ones((K,), jnp.float32)
    zeros_2d = jnp.zeros((RPT, DIM), jnp.float32)

    deg_parts = _deg_kernel(src_deg, zeros_1d, ones_k)
    deg3 = deg_parts.reshape(NC, N_PAD, 1)

    xn = pl.pallas_call(
        _scale_body,
        grid=(N_NODES // BN,),
        in_specs=[
            pl.BlockSpec((NC, BN, 1), lambda i: (0, i, 0)),
            pl.BlockSpec((BN, DIM), lambda i: (i, 0)),
        ],
        out_specs=pl.BlockSpec((BN, DIM), lambda i: (i, 0)),
        out_shape=jax.ShapeDtypeStruct((N_NODES, DIM), jnp.float32),
    )(deg3, x)

    parts = _agg_kernel(src_gat, dst_sc, xn, zeros_2d)

    out = pl.pallas_call(
        _out_body,
        grid=(N_NODES // BN,),
        in_specs=[
            pl.BlockSpec((NC, BN, DIM), lambda i: (0, i, 0)),
            pl.BlockSpec((NC, BN, 1), lambda i: (0, i, 0)),
            pl.BlockSpec((DIM, DIM), lambda i: (0, 0)),
            pl.BlockSpec((DIM,), lambda i: (0,)),
            pl.BlockSpec((DIM, DIM), lambda i: (0, 0)),
            pl.BlockSpec((DIM,), lambda i: (0,)),
        ],
        out_specs=pl.BlockSpec((BN, DIM), lambda i: (i, 0)),
        out_shape=jax.ShapeDtypeStruct((N_NODES, DIM), jnp.float32),
    )(parts, deg3, W1, b1, W2, b2)
    return out
